# baseline (speedup 1.0000x reference)
import functools

import jax
import jax.numpy as jnp
from jax import lax
from jax.experimental import pallas as pl
from jax.experimental.pallas import tpu as pltpu
from jax.experimental.pallas import tpu_sc as plsc

SEQ_LEN, BATCH, DIM = 200, 4096, 64
B_TOTAL = SEQ_LEN * BATCH
NUM_CORES = 2
NUM_SUBCORES = 16
NUM_WORKERS = NUM_CORES * NUM_SUBCORES
B_PER_W = B_TOTAL // NUM_WORKERS
CHUNK = 256
NCHUNK = B_PER_W // CHUNK
NBUF = 4
DEPTH = 3
assert NCHUNK % NBUF == 0


def _make_gather():
    mesh = plsc.VectorSubcoreMesh(core_axis_name="c", subcore_axis_name="s")

    @functools.partial(
        pl.kernel,
        mesh=mesh,
        compiler_params=pltpu.CompilerParams(use_tc_tiling_on_sc=False),
        out_type=jax.ShapeDtypeStruct((B_TOTAL, DIM), jnp.float32),
        scratch_types=[
            pltpu.VMEM((NCHUNK, CHUNK), jnp.int32),
            pltpu.VMEM((NBUF, CHUNK, DIM), jnp.float32),
            pltpu.SemaphoreType.DMA((NBUF,)),
            pltpu.SemaphoreType.DMA((NBUF,)),
        ],
    )
    def gather_kernel(idx_hbm, table_hbm, out_hbm, idx_v, rows_v, gsem, ssem):
        wid = lax.axis_index("s") * NUM_CORES + lax.axis_index("c")
        wbase = wid * B_PER_W

        pltpu.sync_copy(idx_hbm.at[wid], idx_v)

        def start_gather(c, b):
            pltpu.async_copy(table_hbm.at[idx_v.at[c]], rows_v.at[b], gsem.at[b])

        def wait_gather(b):
            pltpu.make_async_copy(
                table_hbm.at[idx_v.at[0]], rows_v.at[b], gsem.at[b]
            ).wait()

        def start_store(c, b):
            pltpu.async_copy(
                rows_v.at[b], out_hbm.at[pl.ds(wbase + c * CHUNK, CHUNK)],
                ssem.at[b],
            )

        def wait_store(b):
            pltpu.make_async_copy(
                rows_v.at[b], out_hbm.at[pl.ds(wbase, CHUNK)], ssem.at[b]
            ).wait()

        for d in range(DEPTH):
            start_gather(d, d)

        def body(i, carry):
            g = i * NBUF
            for b in range(NBUF):
                c = g + b
                nb = (b + DEPTH) % NBUF

                @pl.when(c + DEPTH < NCHUNK)
                def _(c=c, nb=nb):
                    @pl.when(c + DEPTH >= NBUF)
                    def _():
                        wait_store(nb)
                    start_gather(c + DEPTH, nb)

                wait_gather(b)
                start_store(c, b)
            return carry

        lax.fori_loop(0, NCHUNK // NBUF, body, 0)
        for d in range(min(NBUF, NCHUNK)):
            wait_store((NCHUNK - 1 - d) % NBUF)

    return gather_kernel


_gather = _make_gather()


def kernel(input, word_table):
    idx = input.reshape(NUM_WORKERS, NCHUNK, CHUNK)
    out = _gather(idx, word_table)
    return out.reshape(SEQ_LEN, BATCH, DIM)


# speedup vs baseline: 1.0067x; 1.0067x over previous
import functools

import jax
import jax.numpy as jnp
from jax import lax
from jax.experimental import pallas as pl
from jax.experimental.pallas import tpu as pltpu
from jax.experimental.pallas import tpu_sc as plsc

SEQ_LEN, BATCH, DIM = 200, 4096, 64
NUM_CORES = 2
NUM_SUBCORES = 16
NUM_WORKERS = NUM_CORES * NUM_SUBCORES
STRIPE = BATCH // NUM_WORKERS
NBUF = 4
DEPTH = 3
assert SEQ_LEN % NBUF == 0


def _make_gather():
    mesh = plsc.VectorSubcoreMesh(core_axis_name="c", subcore_axis_name="s")

    @functools.partial(
        pl.kernel,
        mesh=mesh,
        compiler_params=pltpu.CompilerParams(use_tc_tiling_on_sc=False),
        out_type=jax.ShapeDtypeStruct((SEQ_LEN, BATCH, DIM), jnp.float32),
        scratch_types=[
            pltpu.VMEM((SEQ_LEN, STRIPE), jnp.int32),
            pltpu.VMEM((NBUF, 1, STRIPE, DIM), jnp.float32),
            pltpu.SemaphoreType.DMA((NBUF,)),
            pltpu.SemaphoreType.DMA((NBUF,)),
        ],
    )
    def gather_kernel(idx_hbm, table_hbm, out_hbm, idx_v, rows_v, gsem, ssem):
        wid = lax.axis_index("s") * NUM_CORES + lax.axis_index("c")
        col0 = wid * STRIPE

        pltpu.sync_copy(idx_hbm.at[:, pl.ds(col0, STRIPE)], idx_v)

        def start_gather(s, b):
            pltpu.async_copy(
                table_hbm.at[idx_v.at[s]], rows_v.at[b, 0], gsem.at[b]
            )

        def wait_gather(b):
            pltpu.make_async_copy(
                table_hbm.at[idx_v.at[0]], rows_v.at[b, 0], gsem.at[b]
            ).wait()

        def start_store(s, b):
            pltpu.async_copy(
                rows_v.at[b],
                out_hbm.at[pl.ds(s, 1), pl.ds(col0, STRIPE), :],
                ssem.at[b],
            )

        def wait_store(b):
            pltpu.make_async_copy(
                rows_v.at[b],
                out_hbm.at[pl.ds(0, 1), pl.ds(col0, STRIPE), :],
                ssem.at[b],
            ).wait()

        for d in range(DEPTH):
            start_gather(d, d)

        def body(i, carry):
            g = i * NBUF
            for b in range(NBUF):
                s = g + b
                nb = (b + DEPTH) % NBUF

                @pl.when(s + DEPTH < SEQ_LEN)
                def _(s=s, nb=nb):
                    @pl.when(s + DEPTH >= NBUF)
                    def _():
                        wait_store(nb)
                    start_gather(s + DEPTH, nb)

                wait_gather(b)
                start_store(s, b)
            return carry

        lax.fori_loop(0, SEQ_LEN // NBUF, body, 0)
        for d in range(min(NBUF, SEQ_LEN)):
            wait_store((SEQ_LEN - 1 - d) % NBUF)

    return gather_kernel


_gather = _make_gather()


def kernel(input, word_table):
    return _gather(input[:, :, 0], word_table)


# speedup vs baseline: 1.3400x; 1.3310x over previous
import functools

import jax
import jax.numpy as jnp
from jax import lax
from jax.experimental import pallas as pl
from jax.experimental.pallas import tpu as pltpu
from jax.experimental.pallas import tpu_sc as plsc

SEQ_LEN, BATCH, DIM = 200, 4096, 64
B_TOTAL = SEQ_LEN * BATCH
OUT_PITCH = 128
NUM_CORES = 2
NUM_SUBCORES = 16
NUM_WORKERS = NUM_CORES * NUM_SUBCORES
STRIPE = BATCH // NUM_WORKERS
NBUF = 4
DEPTH = 3
assert SEQ_LEN % NBUF == 0


def _make_gather():
    mesh = plsc.VectorSubcoreMesh(core_axis_name="c", subcore_axis_name="s")

    @functools.partial(
        pl.kernel,
        mesh=mesh,
        compiler_params=pltpu.CompilerParams(use_tc_tiling_on_sc=False),
        out_type=jax.ShapeDtypeStruct((B_TOTAL, OUT_PITCH), jnp.float32),
        scratch_types=[
            pltpu.VMEM((SEQ_LEN, STRIPE), jnp.int32),
            pltpu.VMEM((NBUF, STRIPE, DIM), jnp.float32),
            pltpu.SemaphoreType.DMA((NBUF,)),
            pltpu.SemaphoreType.DMA((NBUF,)),
        ],
    )
    def gather_kernel(idx_hbm, table_hbm, out_hbm, idx_v, rows_v, gsem, ssem):
        wid = lax.axis_index("s") * NUM_CORES + lax.axis_index("c")
        col0 = wid * STRIPE

        pltpu.sync_copy(idx_hbm.at[:, pl.ds(col0, STRIPE)], idx_v)

        def start_gather(s, b):
            pltpu.async_copy(
                table_hbm.at[idx_v.at[s]], rows_v.at[b], gsem.at[b]
            )

        def wait_gather(b):
            pltpu.make_async_copy(
                table_hbm.at[idx_v.at[0]], rows_v.at[b], gsem.at[b]
            ).wait()

        def start_store(s, b):
            pltpu.async_copy(
                rows_v.at[b],
                out_hbm.at[pl.ds(s * BATCH + col0, STRIPE), pl.ds(0, DIM)],
                ssem.at[b],
            )

        def wait_store(b):
            pltpu.make_async_copy(
                rows_v.at[b],
                out_hbm.at[pl.ds(col0, STRIPE), pl.ds(0, DIM)],
                ssem.at[b],
            ).wait()

        for d in range(DEPTH):
            start_gather(d, d)

        def body(i, carry):
            g = i * NBUF
            for b in range(NBUF):
                s = g + b
                nb = (b + DEPTH) % NBUF

                @pl.when(s + DEPTH < SEQ_LEN)
                def _(s=s, nb=nb):
                    @pl.when(s + DEPTH >= NBUF)
                    def _():
                        wait_store(nb)
                    start_gather(s + DEPTH, nb)

                wait_gather(b)
                start_store(s, b)
            return carry

        lax.fori_loop(0, SEQ_LEN // NBUF, body, 0)
        for d in range(min(NBUF, SEQ_LEN)):
            wait_store((SEQ_LEN - 1 - d) % NBUF)

    return gather_kernel


_gather = _make_gather()


def kernel(input, word_table):
    out = _gather(input[:, :, 0], word_table)
    return out[:, :DIM].reshape(SEQ_LEN, BATCH, DIM)
